# baseline (speedup 1.0000x reference)
import functools

import jax
import jax.numpy as jnp
from jax import lax
from jax.experimental import pallas as pl
from jax.experimental.pallas import tpu as pltpu
from jax.experimental.pallas import tpu_sc as plsc

_PAD = 128


def _sc_gather(mt128, ut128, mids, uids):
    B = mids.shape[0]
    info = plsc.get_sparse_core_info()
    nc, ns = info.num_cores, info.num_subcores
    nw = nc * ns
    b_per_w = B // nw
    mesh = plsc.VectorSubcoreMesh(core_axis_name="c", subcore_axis_name="s")

    @functools.partial(
        pl.kernel,
        mesh=mesh,
        compiler_params=pltpu.CompilerParams(use_tc_tiling_on_sc=True),
        out_type=[
            jax.ShapeDtypeStruct((B, _PAD), jnp.float32),
            jax.ShapeDtypeStruct((B, _PAD), jnp.float32),
        ],
        scratch_types=[
            pltpu.VMEM((b_per_w,), jnp.int32),
            pltpu.VMEM((b_per_w,), jnp.int32),
            pltpu.VMEM((b_per_w // 2, _PAD), jnp.float32),
            pltpu.VMEM((b_per_w // 2, _PAD), jnp.float32),
            pltpu.SemaphoreType.DMA,
            pltpu.SemaphoreType.DMA,
        ],
    )
    def gather_k(mtab, utab, mid, uid, mout, uout,
                 midx_v, uidx_v, buf_a, buf_b, sem_a, sem_b):
        wid = lax.axis_index("s") * nc + lax.axis_index("c")
        base = wid * b_per_w
        half = b_per_w // 2
        pltpu.sync_copy(mid.at[pl.ds(base, b_per_w)], midx_v)
        pltpu.sync_copy(uid.at[pl.ds(base, b_per_w)], uidx_v)
        ma = pltpu.async_copy(mtab.at[midx_v.at[pl.ds(0, half)]], buf_a, sem_a)
        mb = pltpu.async_copy(mtab.at[midx_v.at[pl.ds(half, half)]], buf_b, sem_b)
        ma.wait()
        pltpu.sync_copy(buf_a, mout.at[pl.ds(base, half)])
        ua = pltpu.async_copy(utab.at[uidx_v.at[pl.ds(0, half)]], buf_a, sem_a)
        mb.wait()
        pltpu.sync_copy(buf_b, mout.at[pl.ds(base + half, half)])
        ub = pltpu.async_copy(utab.at[uidx_v.at[pl.ds(half, half)]], buf_b, sem_b)
        ua.wait()
        pltpu.sync_copy(buf_a, uout.at[pl.ds(base, half)])
        ub.wait()
        pltpu.sync_copy(buf_b, uout.at[pl.ds(base + half, half)])

    return gather_k(mt128, ut128, mids, uids)


_GROUP = 25088


def _prep_body(m0, m1, m2, m3, u0, u1, u2, u3, eyes, mo, uo):
    f32 = jnp.float32
    dims = (((0,), (0,)), ((), ()))

    def pack(a, b, c, d):
        return (jax.lax.dot_general(a[...], eyes[0, 0], dims,
                                    preferred_element_type=f32)
                + jax.lax.dot_general(b[...], eyes[0, 1], dims,
                                      preferred_element_type=f32)
                + jax.lax.dot_general(c[...], eyes[0, 2], dims,
                                      preferred_element_type=f32)
                + jax.lax.dot_general(d[...], eyes[0, 3], dims,
                                      preferred_element_type=f32))

    mo[...] = pack(m0, m1, m2, m3)
    uo[...] = pack(u0, u1, u2, u3)


def _prep(movieT, userT, eyes):
    C = 3584
    nb = _GROUP // C
    grid = (nb,)

    def tblock(q):
        return lambda i: (0, q * nb + i)

    return pl.pallas_call(
        _prep_body,
        grid=grid,
        in_specs=(
            [pl.BlockSpec((movieT.shape[0], C), tblock(q)) for q in range(4)]
            + [pl.BlockSpec((userT.shape[0], C), tblock(q)) for q in range(4)]
            + [pl.BlockSpec((1, 4) + eyes.shape[2:], lambda i: (0, 0, 0, 0))]
        ),
        out_specs=[
            pl.BlockSpec((C, _PAD), lambda i: (i, 0)),
            pl.BlockSpec((C, _PAD), lambda i: (i, 0)),
        ],
        out_shape=[
            jax.ShapeDtypeStruct((_GROUP, _PAD), jnp.float32),
            jax.ShapeDtypeStruct((_GROUP, _PAD), jnp.float32),
        ],
    )(movieT, movieT, movieT, movieT, userT, userT, userT, userT, eyes)


def _slot_select(rows, sel, emb):
    s01 = jnp.where(sel == 0, rows[:, 0:emb], rows[:, 32:32 + emb])
    s23 = jnp.where(sel == 2, rows[:, 64:64 + emb], rows[:, 96:96 + emb])
    return jnp.where(sel < 2, s01, s23)


def _mlp_body(mv, msel, us, usel, gnT, vt, wg, bg, w1m, w1u, w1g, w1v, b1,
              w2, b2, w3, b3, w4, b4, out):
    f32 = jnp.float32
    emb = w1m.shape[0]
    dims = (((0,), (0,)), ((), ()))
    g = jax.lax.dot_general(gnT[...], wg[...], dims,
                            preferred_element_type=f32) + bg[...]
    g = jnp.maximum(g, 0.0)
    mve = _slot_select(mv[...], msel[...], emb)
    use = _slot_select(us[...], usel[...], emb)
    x = (jnp.dot(mve, w1m[...], preferred_element_type=f32)
         + jnp.dot(use, w1u[...], preferred_element_type=f32)
         + jnp.dot(g, w1g[...], preferred_element_type=f32)
         + vt[...] * w1v[...]
         + b1[...])
    x = jnp.maximum(x, 0.0)
    x = jnp.maximum(jnp.dot(x, w2[...], preferred_element_type=f32) + b2[...], 0.0)
    x = jnp.maximum(jnp.dot(x, w3[...], preferred_element_type=f32) + b3[...], 0.0)
    x = jnp.maximum(jnp.dot(x, w4[...], preferred_element_type=f32) + b4[...], 0.0)
    m = jnp.max(x, axis=1, keepdims=True)
    e = jnp.exp(x - m)
    out[...] = e / jnp.sum(e, axis=1, keepdims=True)


def _mlp(movieE, msel, userE, usel, genreT, vote, Wg, bg, W1m, W1u, W1g,
         w1v, b1, W2, b2, W3, b3, W4, b4):
    B = movieE.shape[0]
    T = 2048
    grid = (B // T,)

    def btile(minor):
        return pl.BlockSpec((T, minor), lambda i: (i, 0))

    def full(a):
        return pl.BlockSpec(a.shape, lambda i: (0, 0))

    return pl.pallas_call(
        _mlp_body,
        grid=grid,
        in_specs=[
            btile(movieE.shape[1]),
            btile(1),
            btile(userE.shape[1]),
            btile(1),
            pl.BlockSpec((genreT.shape[0], T), lambda i: (0, i)),
            btile(1),
            full(Wg), full(bg), full(W1m), full(W1u), full(W1g),
            full(w1v), full(b1), full(W2), full(b2), full(W3), full(b3),
            full(W4), full(b4),
        ],
        out_specs=btile(5),
        out_shape=jax.ShapeDtypeStruct((B, 5), jnp.float32),
    )(movieE, msel, userE, usel, genreT, vote, Wg, bg, W1m, W1u, W1g, w1v,
      b1, W2, b2, W3, b3, W4, b4)


def kernel(userId, movieId, genre, vote_average, release_date, movie_table,
           user_table, Wg, bg, W1, b1, W2, b2, W3, b3, W4, b4):
    B = userId.shape[0]
    emb = movie_table.shape[1]
    mids = movieId.reshape(B)
    uids = userId.reshape(B)
    eyes = jnp.stack([jnp.eye(emb, _PAD, k=32 * q, dtype=jnp.float32)
                      for q in range(4)])[None]
    mt128, ut128 = _prep(movie_table.T, user_table.T, eyes)
    movieE, userE = _sc_gather(mt128, ut128, mids % _GROUP, uids % _GROUP)
    msel = (mids // _GROUP).astype(jnp.int32).reshape(B, 1)
    usel = (uids // _GROUP).astype(jnp.int32).reshape(B, 1)
    genreT = genre.reshape(B, genre.shape[-1]).T
    W1m = W1[0:20]
    W1u = W1[20:40]
    W1g = W1[40:48]
    w1v = W1[48:49]
    return _mlp(movieE, msel, userE, usel, genreT, vote_average,
                Wg, bg.reshape(1, -1), W1m, W1u, W1g, w1v, b1.reshape(1, -1),
                W2, b2.reshape(1, -1), W3, b3.reshape(1, -1),
                W4, b4.reshape(1, -1))


# speedup vs baseline: 1.4743x; 1.4743x over previous
import functools

import jax
import jax.numpy as jnp
from jax import lax
from jax.experimental import pallas as pl
from jax.experimental.pallas import tpu as pltpu
from jax.experimental.pallas import tpu_sc as plsc

_PAD = 128


def _sc_gather(mt128, ut128, mids, uids, msels, usels):
    B = mids.shape[0]
    info = plsc.get_sparse_core_info()
    nc, ns = info.num_cores, info.num_subcores
    nw = nc * ns
    b_per_w = B // nw
    half = b_per_w // 2
    mesh = plsc.VectorSubcoreMesh(core_axis_name="c", subcore_axis_name="s")

    @functools.partial(
        pl.kernel,
        mesh=mesh,
        compiler_params=pltpu.CompilerParams(use_tc_tiling_on_sc=True, needs_layout_passes=False),
        out_type=[
            jax.ShapeDtypeStruct((B, _PAD), jnp.float32),
            jax.ShapeDtypeStruct((B, _PAD), jnp.float32),
        ],
        scratch_types=[
            pltpu.VMEM((b_per_w,), jnp.int32),
            pltpu.VMEM((b_per_w,), jnp.int32),
            pltpu.VMEM((b_per_w,), jnp.int32),
            pltpu.VMEM((b_per_w,), jnp.int32),
            pltpu.VMEM((half, _PAD), jnp.float32),
            pltpu.VMEM((half, _PAD), jnp.float32),
            pltpu.SemaphoreType.DMA,
            pltpu.SemaphoreType.DMA,
        ],
    )
    def gather_k(mtab, utab, mid, uid, msel, usel, mout, uout,
                 midx_v, uidx_v, msel_v, usel_v, buf_a, buf_b, sem_a, sem_b):
        wid = lax.axis_index("s") * nc + lax.axis_index("c")
        base = wid * b_per_w
        pltpu.sync_copy(mid.at[pl.ds(base, b_per_w)], midx_v)
        pltpu.sync_copy(uid.at[pl.ds(base, b_per_w)], uidx_v)
        pltpu.sync_copy(msel.at[pl.ds(base, b_per_w)], msel_v)
        pltpu.sync_copy(usel.at[pl.ds(base, b_per_w)], usel_v)

        lanes = jnp.arange(16, dtype=jnp.int32)

        def realign(buf, sel_v, off):
            def body(g, _):
                rows = 16 * g + lanes
                sel = sel_v[pl.ds(off + 16 * g, 16)]
                src0 = sel * 32
                for j in range(20):
                    jv = jnp.full((16,), j, dtype=jnp.int32)
                    v = plsc.load_gather(buf, [rows, src0 + jv])
                    plsc.store_scatter(buf, [rows, jv], v)
                return 0
            lax.fori_loop(0, half // 16, body, 0)

        ma = pltpu.async_copy(mtab.at[midx_v.at[pl.ds(0, half)]], buf_a, sem_a)
        mb = pltpu.async_copy(mtab.at[midx_v.at[pl.ds(half, half)]], buf_b, sem_b)
        ma.wait()
        realign(buf_a, msel_v, 0)
        pltpu.sync_copy(buf_a, mout.at[pl.ds(base, half)])
        ua = pltpu.async_copy(utab.at[uidx_v.at[pl.ds(0, half)]], buf_a, sem_a)
        mb.wait()
        realign(buf_b, msel_v, half)
        pltpu.sync_copy(buf_b, mout.at[pl.ds(base + half, half)])
        ub = pltpu.async_copy(utab.at[uidx_v.at[pl.ds(half, half)]], buf_b, sem_b)
        ua.wait()
        realign(buf_a, usel_v, 0)
        pltpu.sync_copy(buf_a, uout.at[pl.ds(base, half)])
        ub.wait()
        realign(buf_b, usel_v, half)
        pltpu.sync_copy(buf_b, uout.at[pl.ds(base + half, half)])

    return gather_k(mt128, ut128, mids, uids, msels, usels)


_GROUP = 25088


def _prep_body(m0, m1, m2, m3, u0, u1, u2, u3, eyes, mo, uo):
    f32 = jnp.float32
    dims = (((0,), (0,)), ((), ()))

    def pack(a, b, c, d):
        return (jax.lax.dot_general(a[...], eyes[0, 0], dims,
                                    preferred_element_type=f32)
                + jax.lax.dot_general(b[...], eyes[0, 1], dims,
                                      preferred_element_type=f32)
                + jax.lax.dot_general(c[...], eyes[0, 2], dims,
                                      preferred_element_type=f32)
                + jax.lax.dot_general(d[...], eyes[0, 3], dims,
                                      preferred_element_type=f32))

    mo[...] = pack(m0, m1, m2, m3)
    uo[...] = pack(u0, u1, u2, u3)


def _prep(movieT, userT, eyes):
    C = 3584
    nb = _GROUP // C
    grid = (nb,)

    def tblock(q):
        return lambda i: (0, q * nb + i)

    return pl.pallas_call(
        _prep_body,
        grid=grid,
        in_specs=(
            [pl.BlockSpec((movieT.shape[0], C), tblock(q)) for q in range(4)]
            + [pl.BlockSpec((userT.shape[0], C), tblock(q)) for q in range(4)]
            + [pl.BlockSpec((1, 4) + eyes.shape[2:], lambda i: (0, 0, 0, 0))]
        ),
        out_specs=[
            pl.BlockSpec((C, _PAD), lambda i: (i, 0)),
            pl.BlockSpec((C, _PAD), lambda i: (i, 0)),
        ],
        out_shape=[
            jax.ShapeDtypeStruct((_GROUP, _PAD), jnp.float32),
            jax.ShapeDtypeStruct((_GROUP, _PAD), jnp.float32),
        ],
    )(movieT, movieT, movieT, movieT, userT, userT, userT, userT, eyes)


def _mlp_body(mv, us, gnT, vt, wg, bg, w1m, w1u, w1g, w1v, b1,
              w2, b2, w3, b3, w4, b4, out):
    f32 = jnp.float32
    emb = w1m.shape[0]
    dims = (((0,), (0,)), ((), ()))
    g = jax.lax.dot_general(gnT[...], wg[...], dims,
                            preferred_element_type=f32) + bg[...]
    g = jnp.maximum(g, 0.0)
    x = (jnp.dot(mv[:, 0:emb], w1m[...], preferred_element_type=f32)
         + jnp.dot(us[:, 0:emb], w1u[...], preferred_element_type=f32)
         + jnp.dot(g, w1g[...], preferred_element_type=f32)
         + jax.lax.dot_general(vt[...], w1v[...], dims,
                               preferred_element_type=f32)
         + b1[...])
    x = jnp.maximum(x, 0.0)
    x = jnp.maximum(jnp.dot(x, w2[...], preferred_element_type=f32) + b2[...], 0.0)
    x = jnp.maximum(jnp.dot(x, w3[...], preferred_element_type=f32) + b3[...], 0.0)
    x = jnp.maximum(jnp.dot(x, w4[...], preferred_element_type=f32) + b4[...], 0.0)
    m = jnp.max(x, axis=1, keepdims=True)
    e = jnp.exp(x - m)
    out[...] = e / jnp.sum(e, axis=1, keepdims=True)


def _mlp(movieE, userE, genreT, vote, Wg, bg, W1m, W1u, W1g,
         w1v, b1, W2, b2, W3, b3, W4, b4):
    B = movieE.shape[0]
    T = 2048
    grid = (B // T,)

    def btile(minor):
        return pl.BlockSpec((T, minor), lambda i: (i, 0))

    def full(a):
        return pl.BlockSpec(a.shape, lambda i: (0, 0))

    return pl.pallas_call(
        _mlp_body,
        grid=grid,
        in_specs=[
            btile(movieE.shape[1]),
            btile(userE.shape[1]),
            pl.BlockSpec((genreT.shape[0], T), lambda i: (0, i)),
            pl.BlockSpec((1, T), lambda i: (0, i)),
            full(Wg), full(bg), full(W1m), full(W1u), full(W1g),
            full(w1v), full(b1), full(W2), full(b2), full(W3), full(b3),
            full(W4), full(b4),
        ],
        out_specs=btile(5),
        out_shape=jax.ShapeDtypeStruct((B, 5), jnp.float32),
    )(movieE, userE, genreT, vote, Wg, bg, W1m, W1u, W1g, w1v,
      b1, W2, b2, W3, b3, W4, b4)


def kernel(userId, movieId, genre, vote_average, release_date, movie_table,
           user_table, Wg, bg, W1, b1, W2, b2, W3, b3, W4, b4):
    B = userId.shape[0]
    emb = movie_table.shape[1]
    mids = movieId.reshape(B)
    uids = userId.reshape(B)
    eyes = jnp.stack([jnp.eye(emb, _PAD, k=32 * q, dtype=jnp.float32)
                      for q in range(4)])[None]
    mt128, ut128 = _prep(movie_table.T, user_table.T, eyes)
    g32 = jnp.int32(_GROUP)
    movieE, userE = _sc_gather(mt128, ut128,
                               jax.lax.rem(mids, g32), jax.lax.rem(uids, g32),
                               jax.lax.div(mids, g32), jax.lax.div(uids, g32))
    genreT = genre.reshape(B, genre.shape[-1]).T
    W1m = W1[0:20]
    W1u = W1[20:40]
    W1g = W1[40:48]
    w1v = W1[48:49]
    return _mlp(movieE, userE, genreT, vote_average.T,
                Wg, bg.reshape(1, -1), W1m, W1u, W1g, w1v, b1.reshape(1, -1),
                W2, b2.reshape(1, -1), W3, b3.reshape(1, -1),
                W4, b4.reshape(1, -1))


# speedup vs baseline: 1.4804x; 1.0041x over previous
import functools

import jax
import jax.numpy as jnp
from jax import lax
from jax.experimental import pallas as pl
from jax.experimental.pallas import tpu as pltpu
from jax.experimental.pallas import tpu_sc as plsc

_PAD = 128


def _sc_gather(mt128, ut128, mids, uids, msels, usels):
    B = mids.shape[0]
    info = plsc.get_sparse_core_info()
    nc, ns = info.num_cores, info.num_subcores
    nw = nc * ns
    b_per_w = B // nw
    half = b_per_w // 2
    mesh = plsc.VectorSubcoreMesh(core_axis_name="c", subcore_axis_name="s")

    @functools.partial(
        pl.kernel,
        mesh=mesh,
        compiler_params=pltpu.CompilerParams(use_tc_tiling_on_sc=True, needs_layout_passes=False),
        out_type=[
            jax.ShapeDtypeStruct((B, _PAD), jnp.float32),
            jax.ShapeDtypeStruct((B, _PAD), jnp.float32),
        ],
        scratch_types=[
            pltpu.VMEM((b_per_w,), jnp.int32),
            pltpu.VMEM((b_per_w,), jnp.int32),
            pltpu.VMEM((b_per_w,), jnp.int32),
            pltpu.VMEM((b_per_w,), jnp.int32),
            pltpu.VMEM((half, _PAD), jnp.float32),
            pltpu.VMEM((half, _PAD), jnp.float32),
            pltpu.SemaphoreType.DMA,
            pltpu.SemaphoreType.DMA,
            pltpu.SemaphoreType.DMA,
            pltpu.SemaphoreType.DMA,
        ],
    )
    def gather_k(mtab, utab, mid, uid, msel, usel, mout, uout,
                 midx_v, uidx_v, msel_v, usel_v, buf_a, buf_b,
                 sem_a, sem_b, sem_oa, sem_ob):
        wid = lax.axis_index("s") * nc + lax.axis_index("c")
        base = wid * b_per_w
        pltpu.sync_copy(mid.at[pl.ds(base, b_per_w)], midx_v)
        pltpu.sync_copy(uid.at[pl.ds(base, b_per_w)], uidx_v)

        lanes = jnp.arange(16, dtype=jnp.int32)

        def realign(buf, sel_v, off):
            def body(g, _):
                rows = 16 * g + lanes
                sel = sel_v[pl.ds(off + 16 * g, 16)]
                src0 = sel * 32
                for j in range(20):
                    jv = jnp.full((16,), j, dtype=jnp.int32)
                    v = plsc.load_gather(buf, [rows, src0 + jv])
                    plsc.store_scatter(buf, [rows, jv], v)
                return 0
            lax.fori_loop(0, half // 16, body, 0)

        ma = pltpu.async_copy(mtab.at[midx_v.at[pl.ds(0, half)]], buf_a, sem_a)
        mb = pltpu.async_copy(mtab.at[midx_v.at[pl.ds(half, half)]], buf_b, sem_b)
        pltpu.sync_copy(msel.at[pl.ds(base, b_per_w)], msel_v)
        pltpu.sync_copy(usel.at[pl.ds(base, b_per_w)], usel_v)
        ma.wait()
        realign(buf_a, msel_v, 0)
        oa = pltpu.async_copy(buf_a, mout.at[pl.ds(base, half)], sem_oa)
        mb.wait()
        realign(buf_b, msel_v, half)
        ob = pltpu.async_copy(buf_b, mout.at[pl.ds(base + half, half)], sem_ob)
        oa.wait()
        ua = pltpu.async_copy(utab.at[uidx_v.at[pl.ds(0, half)]], buf_a, sem_a)
        ob.wait()
        ub = pltpu.async_copy(utab.at[uidx_v.at[pl.ds(half, half)]], buf_b, sem_b)
        ua.wait()
        realign(buf_a, usel_v, 0)
        oa = pltpu.async_copy(buf_a, uout.at[pl.ds(base, half)], sem_oa)
        ub.wait()
        realign(buf_b, usel_v, half)
        ob = pltpu.async_copy(buf_b, uout.at[pl.ds(base + half, half)], sem_ob)
        oa.wait()
        ob.wait()

    return gather_k(mt128, ut128, mids, uids, msels, usels)


_GROUP = 25088


def _prep_body(m0, m1, m2, m3, u0, u1, u2, u3, eyes, mo, uo):
    f32 = jnp.float32
    dims = (((0,), (0,)), ((), ()))

    def pack(a, b, c, d):
        return (jax.lax.dot_general(a[...], eyes[0, 0], dims,
                                    preferred_element_type=f32)
                + jax.lax.dot_general(b[...], eyes[0, 1], dims,
                                      preferred_element_type=f32)
                + jax.lax.dot_general(c[...], eyes[0, 2], dims,
                                      preferred_element_type=f32)
                + jax.lax.dot_general(d[...], eyes[0, 3], dims,
                                      preferred_element_type=f32))

    mo[...] = pack(m0, m1, m2, m3)
    uo[...] = pack(u0, u1, u2, u3)


def _prep(movieT, userT, eyes):
    C = 3584
    nb = _GROUP // C
    grid = (nb,)

    def tblock(q):
        return lambda i: (0, q * nb + i)

    return pl.pallas_call(
        _prep_body,
        grid=grid,
        in_specs=(
            [pl.BlockSpec((movieT.shape[0], C), tblock(q)) for q in range(4)]
            + [pl.BlockSpec((userT.shape[0], C), tblock(q)) for q in range(4)]
            + [pl.BlockSpec((1, 4) + eyes.shape[2:], lambda i: (0, 0, 0, 0))]
        ),
        out_specs=[
            pl.BlockSpec((C, _PAD), lambda i: (i, 0)),
            pl.BlockSpec((C, _PAD), lambda i: (i, 0)),
        ],
        out_shape=[
            jax.ShapeDtypeStruct((_GROUP, _PAD), jnp.float32),
            jax.ShapeDtypeStruct((_GROUP, _PAD), jnp.float32),
        ],
    )(movieT, movieT, movieT, movieT, userT, userT, userT, userT, eyes)


def _mlp_body(mv, us, gnT, vt, wg, bg, w1m, w1u, w1g, w1v, b1,
              w2, b2, w3, b3, w4, b4, out):
    f32 = jnp.float32
    emb = w1m.shape[0]
    dims = (((0,), (0,)), ((), ()))
    g = jax.lax.dot_general(gnT[...], wg[...], dims,
                            preferred_element_type=f32) + bg[...]
    g = jnp.maximum(g, 0.0)
    x = (jnp.dot(mv[:, 0:emb], w1m[...], preferred_element_type=f32)
         + jnp.dot(us[:, 0:emb], w1u[...], preferred_element_type=f32)
         + jnp.dot(g, w1g[...], preferred_element_type=f32)
         + jax.lax.dot_general(vt[...], w1v[...], dims,
                               preferred_element_type=f32)
         + b1[...])
    x = jnp.maximum(x, 0.0)
    x = jnp.maximum(jnp.dot(x, w2[...], preferred_element_type=f32) + b2[...], 0.0)
    x = jnp.maximum(jnp.dot(x, w3[...], preferred_element_type=f32) + b3[...], 0.0)
    x = jnp.maximum(jnp.dot(x, w4[...], preferred_element_type=f32) + b4[...], 0.0)
    m = jnp.max(x, axis=1, keepdims=True)
    e = jnp.exp(x - m)
    out[...] = e / jnp.sum(e, axis=1, keepdims=True)


def _mlp(movieE, userE, genreT, vote, Wg, bg, W1m, W1u, W1g,
         w1v, b1, W2, b2, W3, b3, W4, b4):
    B = movieE.shape[0]
    T = 2048
    grid = (B // T,)

    def btile(minor):
        return pl.BlockSpec((T, minor), lambda i: (i, 0))

    def full(a):
        return pl.BlockSpec(a.shape, lambda i: (0, 0))

    return pl.pallas_call(
        _mlp_body,
        grid=grid,
        in_specs=[
            btile(movieE.shape[1]),
            btile(userE.shape[1]),
            pl.BlockSpec((genreT.shape[0], T), lambda i: (0, i)),
            pl.BlockSpec((1, T), lambda i: (0, i)),
            full(Wg), full(bg), full(W1m), full(W1u), full(W1g),
            full(w1v), full(b1), full(W2), full(b2), full(W3), full(b3),
            full(W4), full(b4),
        ],
        out_specs=btile(5),
        out_shape=jax.ShapeDtypeStruct((B, 5), jnp.float32),
    )(movieE, userE, genreT, vote, Wg, bg, W1m, W1u, W1g, w1v,
      b1, W2, b2, W3, b3, W4, b4)


def kernel(userId, movieId, genre, vote_average, release_date, movie_table,
           user_table, Wg, bg, W1, b1, W2, b2, W3, b3, W4, b4):
    B = userId.shape[0]
    emb = movie_table.shape[1]
    mids = movieId.reshape(B)
    uids = userId.reshape(B)
    eyes = jnp.stack([jnp.eye(emb, _PAD, k=32 * q, dtype=jnp.float32)
                      for q in range(4)])[None]
    mt128, ut128 = _prep(movie_table.T, user_table.T, eyes)
    g32 = jnp.int32(_GROUP)
    movieE, userE = _sc_gather(mt128, ut128,
                               jax.lax.rem(mids, g32), jax.lax.rem(uids, g32),
                               jax.lax.div(mids, g32), jax.lax.div(uids, g32))
    genreT = genre.reshape(B, genre.shape[-1]).T
    W1m = W1[0:20]
    W1u = W1[20:40]
    W1g = W1[40:48]
    w1v = W1[48:49]
    return _mlp(movieE, userE, genreT, vote_average.T,
                Wg, bg.reshape(1, -1), W1m, W1u, W1g, w1v, b1.reshape(1, -1),
                W2, b2.reshape(1, -1), W3, b3.reshape(1, -1),
                W4, b4.reshape(1, -1))


# speedup vs baseline: 1.6277x; 1.0995x over previous
import functools

import jax
import jax.numpy as jnp
from jax import lax
from jax.experimental import pallas as pl
from jax.experimental.pallas import tpu as pltpu
from jax.experimental.pallas import tpu_sc as plsc

_PAD = 128


def _sc_gather(mt128, ut128, mids, uids, msels, usels):
    B = mids.shape[0]
    info = plsc.get_sparse_core_info()
    nc, ns = info.num_cores, info.num_subcores
    nw = nc * ns
    b_per_w = B // nw
    half = b_per_w // 2
    mesh = plsc.VectorSubcoreMesh(core_axis_name="c", subcore_axis_name="s")

    @functools.partial(
        pl.kernel,
        mesh=mesh,
        compiler_params=pltpu.CompilerParams(use_tc_tiling_on_sc=True, needs_layout_passes=False),
        out_type=[
            jax.ShapeDtypeStruct((B, _PAD), jnp.float32),
            jax.ShapeDtypeStruct((B, _PAD), jnp.float32),
        ],
        scratch_types=[
            pltpu.VMEM((b_per_w,), jnp.int32),
            pltpu.VMEM((b_per_w,), jnp.int32),
            pltpu.VMEM((b_per_w,), jnp.int32),
            pltpu.VMEM((b_per_w,), jnp.int32),
            pltpu.VMEM((half, _PAD), jnp.float32),
            pltpu.VMEM((half, _PAD), jnp.float32),
            pltpu.SemaphoreType.DMA,
            pltpu.SemaphoreType.DMA,
            pltpu.SemaphoreType.DMA,
            pltpu.SemaphoreType.DMA,
        ],
    )
    def gather_k(mtab, utab, mid, uid, msel, usel, mout, uout,
                 midx_v, uidx_v, msel_v, usel_v, buf_a, buf_b,
                 sem_a, sem_b, sem_oa, sem_ob):
        wid = lax.axis_index("s") * nc + lax.axis_index("c")
        base = wid * b_per_w
        pltpu.sync_copy(mid.at[pl.ds(base, b_per_w)], midx_v)
        pltpu.sync_copy(uid.at[pl.ds(base, b_per_w)], uidx_v)

        lanes = jnp.arange(16, dtype=jnp.int32)
        lane120 = jnp.full((16,), 120, dtype=jnp.int32)

        def write_sel(buf, sel_v, off):
            def body(g, _):
                rows = 16 * g + lanes
                sel = sel_v[pl.ds(off + 16 * g, 16)]
                plsc.store_scatter(buf, [rows, lane120],
                                   sel.astype(jnp.float32))
                return 0
            lax.fori_loop(0, half // 16, body, 0)

        ma = pltpu.async_copy(mtab.at[midx_v.at[pl.ds(0, half)]], buf_a, sem_a)
        mb = pltpu.async_copy(mtab.at[midx_v.at[pl.ds(half, half)]], buf_b, sem_b)
        pltpu.sync_copy(msel.at[pl.ds(base, b_per_w)], msel_v)
        pltpu.sync_copy(usel.at[pl.ds(base, b_per_w)], usel_v)
        ma.wait()
        write_sel(buf_a, msel_v, 0)
        oa = pltpu.async_copy(buf_a, mout.at[pl.ds(base, half)], sem_oa)
        mb.wait()
        write_sel(buf_b, msel_v, half)
        ob = pltpu.async_copy(buf_b, mout.at[pl.ds(base + half, half)], sem_ob)
        oa.wait()
        ua = pltpu.async_copy(utab.at[uidx_v.at[pl.ds(0, half)]], buf_a, sem_a)
        ob.wait()
        ub = pltpu.async_copy(utab.at[uidx_v.at[pl.ds(half, half)]], buf_b, sem_b)
        ua.wait()
        write_sel(buf_a, usel_v, 0)
        oa = pltpu.async_copy(buf_a, uout.at[pl.ds(base, half)], sem_oa)
        ub.wait()
        write_sel(buf_b, usel_v, half)
        ob = pltpu.async_copy(buf_b, uout.at[pl.ds(base + half, half)], sem_ob)
        oa.wait()
        ob.wait()

    return gather_k(mt128, ut128, mids, uids, msels, usels)


_GROUP = 50176


def _prep_body(m0, m1, u0, u1, eyes, mo, uo):
    f32 = jnp.float32
    dims = (((0,), (0,)), ((), ()))

    def pack(a, b):
        return (jax.lax.dot_general(a[...], eyes[0, 0], dims,
                                    preferred_element_type=f32)
                + jax.lax.dot_general(b[...], eyes[0, 1], dims,
                                      preferred_element_type=f32))

    mo[...] = pack(m0, m1)
    uo[...] = pack(u0, u1)


def _prep(movieT, userT, eyes):
    C = 3584
    nb = _GROUP // C
    grid = (nb,)

    def tblock(q):
        return lambda i: (0, q * nb + i)

    return pl.pallas_call(
        _prep_body,
        grid=grid,
        in_specs=(
            [pl.BlockSpec((movieT.shape[0], C), tblock(q)) for q in range(2)]
            + [pl.BlockSpec((userT.shape[0], C), tblock(q)) for q in range(2)]
            + [pl.BlockSpec((1, 2) + eyes.shape[2:], lambda i: (0, 0, 0, 0))]
        ),
        out_specs=[
            pl.BlockSpec((C, _PAD), lambda i: (i, 0)),
            pl.BlockSpec((C, _PAD), lambda i: (i, 0)),
        ],
        out_shape=[
            jax.ShapeDtypeStruct((_GROUP, _PAD), jnp.float32),
            jax.ShapeDtypeStruct((_GROUP, _PAD), jnp.float32),
        ],
    )(movieT, movieT, userT, userT, eyes)


def _mlp_body(mv, us, gnT, vt, wg, bg, w1m, w1u, w1g, w1v, b1,
              w2, b2, w3, b3, w4, b4, out):
    f32 = jnp.float32
    emb = w1m.shape[0]
    dims = (((0,), (0,)), ((), ()))
    g = jax.lax.dot_general(gnT[...], wg[...], dims,
                            preferred_element_type=f32) + bg[...]
    g = jnp.maximum(g, 0.0)
    mvr = mv[...]
    usr = us[...]
    mve = jnp.where(mvr[:, 120:121] < 0.5, mvr[:, 0:emb], mvr[:, 64:64 + emb])
    use = jnp.where(usr[:, 120:121] < 0.5, usr[:, 0:emb], usr[:, 64:64 + emb])
    x = (jnp.dot(mve, w1m[...], preferred_element_type=f32)
         + jnp.dot(use, w1u[...], preferred_element_type=f32)
         + jnp.dot(g, w1g[...], preferred_element_type=f32)
         + jax.lax.dot_general(vt[...], w1v[...], dims,
                               preferred_element_type=f32)
         + b1[...])
    x = jnp.maximum(x, 0.0)
    x = jnp.maximum(jnp.dot(x, w2[...], preferred_element_type=f32) + b2[...], 0.0)
    x = jnp.maximum(jnp.dot(x, w3[...], preferred_element_type=f32) + b3[...], 0.0)
    x = jnp.maximum(jnp.dot(x, w4[...], preferred_element_type=f32) + b4[...], 0.0)
    m = jnp.max(x, axis=1, keepdims=True)
    e = jnp.exp(x - m)
    out[...] = e / jnp.sum(e, axis=1, keepdims=True)


def _mlp(movieE, userE, genreT, vote, Wg, bg, W1m, W1u, W1g,
         w1v, b1, W2, b2, W3, b3, W4, b4):
    B = movieE.shape[0]
    T = 2048
    grid = (B // T,)

    def btile(minor):
        return pl.BlockSpec((T, minor), lambda i: (i, 0))

    def full(a):
        return pl.BlockSpec(a.shape, lambda i: (0, 0))

    return pl.pallas_call(
        _mlp_body,
        grid=grid,
        in_specs=[
            btile(movieE.shape[1]),
            btile(userE.shape[1]),
            pl.BlockSpec((genreT.shape[0], T), lambda i: (0, i)),
            pl.BlockSpec((1, T), lambda i: (0, i)),
            full(Wg), full(bg), full(W1m), full(W1u), full(W1g),
            full(w1v), full(b1), full(W2), full(b2), full(W3), full(b3),
            full(W4), full(b4),
        ],
        out_specs=btile(5),
        out_shape=jax.ShapeDtypeStruct((B, 5), jnp.float32),
    )(movieE, userE, genreT, vote, Wg, bg, W1m, W1u, W1g, w1v,
      b1, W2, b2, W3, b3, W4, b4)


def kernel(userId, movieId, genre, vote_average, release_date, movie_table,
           user_table, Wg, bg, W1, b1, W2, b2, W3, b3, W4, b4):
    B = userId.shape[0]
    emb = movie_table.shape[1]
    mids = movieId.reshape(B)
    uids = userId.reshape(B)
    eyes = jnp.stack([jnp.eye(emb, _PAD, k=64 * q, dtype=jnp.float32)
                      for q in range(2)])[None]
    mt128, ut128 = _prep(movie_table.T, user_table.T, eyes)
    g32 = jnp.int32(_GROUP)
    movieE, userE = _sc_gather(mt128, ut128,
                               jax.lax.rem(mids, g32), jax.lax.rem(uids, g32),
                               jax.lax.div(mids, g32), jax.lax.div(uids, g32))
    genreT = genre.reshape(B, genre.shape[-1]).T
    W1m = W1[0:20]
    W1u = W1[20:40]
    W1g = W1[40:48]
    w1v = W1[48:49]
    return _mlp(movieE, userE, genreT, vote_average.T,
                Wg, bg.reshape(1, -1), W1m, W1u, W1g, w1v, b1.reshape(1, -1),
                W2, b2.reshape(1, -1), W3, b3.reshape(1, -1),
                W4, b4.reshape(1, -1))


# speedup vs baseline: 1.7848x; 1.0965x over previous
import functools

import jax
import jax.numpy as jnp
from jax import lax
from jax.experimental import pallas as pl
from jax.experimental.pallas import tpu as pltpu
from jax.experimental.pallas import tpu_sc as plsc

_PAD = 128


def _sc_gather(mt128, ut128, mids, uids, msels, usels):
    B = mids.shape[0]
    info = plsc.get_sparse_core_info()
    nc, ns = info.num_cores, info.num_subcores
    nw = nc * ns
    b_per_w = B // nw
    half = b_per_w // 2
    mesh = plsc.VectorSubcoreMesh(core_axis_name="c", subcore_axis_name="s")

    @functools.partial(
        pl.kernel,
        mesh=mesh,
        compiler_params=pltpu.CompilerParams(use_tc_tiling_on_sc=True, needs_layout_passes=False),
        out_type=[
            jax.ShapeDtypeStruct((B, _PAD), jnp.float32),
            jax.ShapeDtypeStruct((B, _PAD), jnp.float32),
        ],
        scratch_types=[
            pltpu.VMEM((b_per_w,), jnp.int32),
            pltpu.VMEM((b_per_w,), jnp.int32),
            pltpu.VMEM((b_per_w,), jnp.int32),
            pltpu.VMEM((b_per_w,), jnp.int32),
            pltpu.VMEM((half, _PAD), jnp.float32),
            pltpu.VMEM((half, _PAD), jnp.float32),
            pltpu.SemaphoreType.DMA,
            pltpu.SemaphoreType.DMA,
            pltpu.SemaphoreType.DMA,
            pltpu.SemaphoreType.DMA,
        ],
    )
    def gather_k(mtab, utab, mid, uid, msel, usel, mout, uout,
                 midx_v, uidx_v, msel_v, usel_v, buf_a, buf_b,
                 sem_a, sem_b, sem_oa, sem_ob):
        wid = lax.axis_index("s") * nc + lax.axis_index("c")
        base = wid * b_per_w
        pltpu.sync_copy(mid.at[pl.ds(base, b_per_w)], midx_v)
        pltpu.sync_copy(uid.at[pl.ds(base, b_per_w)], uidx_v)

        lanes = jnp.arange(16, dtype=jnp.int32)
        lane120 = jnp.full((16,), 120, dtype=jnp.int32)

        def write_sel(buf, sel_v, off):
            def body(g, _):
                rows = 16 * g + lanes
                sel = sel_v[pl.ds(off + 16 * g, 16)]
                plsc.store_scatter(buf, [rows, lane120],
                                   sel.astype(jnp.float32))
                return 0
            lax.fori_loop(0, half // 16, body, 0)

        ma = pltpu.async_copy(mtab.at[midx_v.at[pl.ds(0, half)]], buf_a, sem_a)
        mb = pltpu.async_copy(mtab.at[midx_v.at[pl.ds(half, half)]], buf_b, sem_b)
        pltpu.sync_copy(msel.at[pl.ds(base, b_per_w)], msel_v)
        pltpu.sync_copy(usel.at[pl.ds(base, b_per_w)], usel_v)
        ma.wait()
        write_sel(buf_a, msel_v, 0)
        oa = pltpu.async_copy(buf_a, mout.at[pl.ds(base, half)], sem_oa)
        mb.wait()
        write_sel(buf_b, msel_v, half)
        ob = pltpu.async_copy(buf_b, mout.at[pl.ds(base + half, half)], sem_ob)
        oa.wait()
        ua = pltpu.async_copy(utab.at[uidx_v.at[pl.ds(0, half)]], buf_a, sem_a)
        ob.wait()
        ub = pltpu.async_copy(utab.at[uidx_v.at[pl.ds(half, half)]], buf_b, sem_b)
        ua.wait()
        write_sel(buf_a, usel_v, 0)
        oa = pltpu.async_copy(buf_a, uout.at[pl.ds(base, half)], sem_oa)
        ub.wait()
        write_sel(buf_b, usel_v, half)
        ob = pltpu.async_copy(buf_b, uout.at[pl.ds(base + half, half)], sem_ob)
        oa.wait()
        ob.wait()

    return gather_k(mt128, ut128, mids, uids, msels, usels)


_GROUP = 50176


def _prep_body(m0, m1, u0, u1, eyes, mo, uo):
    f32 = jnp.float32
    dims = (((0,), (0,)), ((), ()))

    def pack(a, b):
        return (jax.lax.dot_general(a[...], eyes[0, 0], dims,
                                    preferred_element_type=f32)
                + jax.lax.dot_general(b[...], eyes[0, 1], dims,
                                      preferred_element_type=f32))

    mo[...] = pack(m0, m1)
    uo[...] = pack(u0, u1)


def _prep(movieT, userT, eyes):
    C = 3584
    nb = _GROUP // C
    grid = (nb,)

    def tblock(q):
        return lambda i: (0, q * nb + i)

    return pl.pallas_call(
        _prep_body,
        grid=grid,
        in_specs=(
            [pl.BlockSpec((movieT.shape[0], C), tblock(q)) for q in range(2)]
            + [pl.BlockSpec((userT.shape[0], C), tblock(q)) for q in range(2)]
            + [pl.BlockSpec((1, 2) + eyes.shape[2:], lambda i: (0, 0, 0, 0))]
        ),
        out_specs=[
            pl.BlockSpec((C, _PAD), lambda i: (i, 0)),
            pl.BlockSpec((C, _PAD), lambda i: (i, 0)),
        ],
        out_shape=[
            jax.ShapeDtypeStruct((_GROUP, _PAD), jnp.float32),
            jax.ShapeDtypeStruct((_GROUP, _PAD), jnp.float32),
        ],
    )(movieT, movieT, userT, userT, eyes)


def _mlp_body(mv, us, gnT, vt, wg, bg, w1m, w1u, w1g, w1v, b1,
              w2, b2, w3, b3, w4, b4, out):
    f32 = jnp.float32
    emb = w1m.shape[0]
    dims = (((0,), (0,)), ((), ()))
    g = jax.lax.dot_general(gnT[...], wg[...], dims,
                            preferred_element_type=f32) + bg[...]
    g = jnp.maximum(g, 0.0)
    mvr = mv[...]
    usr = us[...]
    mve = jnp.where(mvr[:, 120:121] < 0.5, mvr[:, 0:emb], mvr[:, 64:64 + emb])
    use = jnp.where(usr[:, 120:121] < 0.5, usr[:, 0:emb], usr[:, 64:64 + emb])
    x = (jnp.dot(mve, w1m[...], preferred_element_type=f32)
         + jnp.dot(use, w1u[...], preferred_element_type=f32)
         + jnp.dot(g, w1g[...], preferred_element_type=f32)
         + jax.lax.dot_general(vt[...], w1v[...], dims,
                               preferred_element_type=f32)
         + b1[...])
    x = jnp.maximum(x, 0.0)
    x = jnp.maximum(jnp.dot(x, w2[...], preferred_element_type=f32) + b2[...], 0.0)
    x = jnp.maximum(jnp.dot(x, w3[...], preferred_element_type=f32) + b3[...], 0.0)
    dt = (((0,), (1,)), ((), ()))
    xt = jnp.maximum(jax.lax.dot_general(w4[...], x, dt,
                                         preferred_element_type=f32)
                     + b4[...], 0.0)
    m = jnp.max(xt, axis=0, keepdims=True)
    e = jnp.exp(xt - m)
    out[...] = e / jnp.sum(e, axis=0, keepdims=True)


def _mlp(movieE, userE, genreT, vote, Wg, bg, W1m, W1u, W1g,
         w1v, b1, W2, b2, W3, b3, W4, b4):
    B = movieE.shape[0]
    T = 2048
    grid = (B // T,)

    def btile(minor):
        return pl.BlockSpec((T, minor), lambda i: (i, 0))

    def full(a):
        return pl.BlockSpec(a.shape, lambda i: (0, 0))

    return pl.pallas_call(
        _mlp_body,
        grid=grid,
        in_specs=[
            btile(movieE.shape[1]),
            btile(userE.shape[1]),
            pl.BlockSpec((genreT.shape[0], T), lambda i: (0, i)),
            pl.BlockSpec((1, T), lambda i: (0, i)),
            full(Wg), full(bg), full(W1m), full(W1u), full(W1g),
            full(w1v), full(b1), full(W2), full(b2), full(W3), full(b3),
            full(W4), full(b4),
        ],
        out_specs=pl.BlockSpec((5, T), lambda i: (0, i)),
        out_shape=jax.ShapeDtypeStruct((5, B), jnp.float32),
    )(movieE, userE, genreT, vote, Wg, bg, W1m, W1u, W1g, w1v,
      b1, W2, b2, W3, b3, W4, b4)


def kernel(userId, movieId, genre, vote_average, release_date, movie_table,
           user_table, Wg, bg, W1, b1, W2, b2, W3, b3, W4, b4):
    B = userId.shape[0]
    emb = movie_table.shape[1]
    mids = movieId.reshape(B)
    uids = userId.reshape(B)
    eyes = jnp.stack([jnp.eye(emb, _PAD, k=64 * q, dtype=jnp.float32)
                      for q in range(2)])[None]
    mt128, ut128 = _prep(movie_table.T, user_table.T, eyes)
    g32 = jnp.int32(_GROUP)
    movieE, userE = _sc_gather(mt128, ut128,
                               jax.lax.rem(mids, g32), jax.lax.rem(uids, g32),
                               jax.lax.div(mids, g32), jax.lax.div(uids, g32))
    genreT = genre.reshape(B, genre.shape[-1]).T
    W1m = W1[0:20]
    W1u = W1[20:40]
    W1g = W1[40:48]
    w1v = W1[48:49]
    return _mlp(movieE, userE, genreT, vote_average.T,
                Wg, bg.reshape(1, -1), W1m, W1u, W1g, w1v, b1.reshape(1, -1),
                W2, b2.reshape(1, -1), W3, b3.reshape(1, -1),
                W4, b4.reshape(-1, 1)).T


# speedup vs baseline: 1.8597x; 1.0420x over previous
import functools

import jax
import jax.numpy as jnp
from jax import lax
from jax.experimental import pallas as pl
from jax.experimental.pallas import tpu as pltpu
from jax.experimental.pallas import tpu_sc as plsc

_PAD = 128


def _sc_gather(mt128, ut128, mids, uids):
    B = mids.shape[0]
    info = plsc.get_sparse_core_info()
    nc, ns = info.num_cores, info.num_subcores
    nw = nc * ns
    b_per_w = B // nw
    half = b_per_w // 2
    mesh = plsc.VectorSubcoreMesh(core_axis_name="c", subcore_axis_name="s")

    @functools.partial(
        pl.kernel,
        mesh=mesh,
        compiler_params=pltpu.CompilerParams(use_tc_tiling_on_sc=True, needs_layout_passes=False),
        out_type=[
            jax.ShapeDtypeStruct((B, _PAD), jnp.float32),
            jax.ShapeDtypeStruct((B, _PAD), jnp.float32),
        ],
        scratch_types=[
            pltpu.VMEM((b_per_w,), jnp.int32),
            pltpu.VMEM((b_per_w,), jnp.int32),
            pltpu.VMEM((b_per_w,), jnp.int32),
            pltpu.VMEM((b_per_w,), jnp.int32),
            pltpu.VMEM((half, _PAD), jnp.float32),
            pltpu.VMEM((half, _PAD), jnp.float32),
            pltpu.SemaphoreType.DMA,
            pltpu.SemaphoreType.DMA,
            pltpu.SemaphoreType.DMA,
            pltpu.SemaphoreType.DMA,
        ],
    )
    def gather_k(mtab, utab, mid, uid, mout, uout,
                 midx_v, uidx_v, msel_v, usel_v, buf_a, buf_b,
                 sem_a, sem_b, sem_oa, sem_ob):
        wid = lax.axis_index("s") * nc + lax.axis_index("c")
        base = wid * b_per_w
        pltpu.sync_copy(mid.at[pl.ds(base, b_per_w)], midx_v)
        pltpu.sync_copy(uid.at[pl.ds(base, b_per_w)], uidx_v)
        g32 = jnp.int32(_GROUP)

        def split_ids(k, _):
            mi = midx_v[pl.ds(16 * k, 16)]
            ui = uidx_v[pl.ds(16 * k, 16)]
            msel_v[pl.ds(16 * k, 16)] = lax.div(mi, g32)
            usel_v[pl.ds(16 * k, 16)] = lax.div(ui, g32)
            midx_v[pl.ds(16 * k, 16)] = lax.rem(mi, g32)
            uidx_v[pl.ds(16 * k, 16)] = lax.rem(ui, g32)
            return 0

        lax.fori_loop(0, b_per_w // 16, split_ids, 0)

        lanes = jnp.arange(16, dtype=jnp.int32)
        lane120 = jnp.full((16,), 120, dtype=jnp.int32)

        def write_sel(buf, sel_v, off):
            def body(g, _):
                rows = 16 * g + lanes
                sel = sel_v[pl.ds(off + 16 * g, 16)]
                plsc.store_scatter(buf, [rows, lane120],
                                   sel.astype(jnp.float32))
                return 0
            lax.fori_loop(0, half // 16, body, 0)

        ma = pltpu.async_copy(mtab.at[midx_v.at[pl.ds(0, half)]], buf_a, sem_a)
        mb = pltpu.async_copy(mtab.at[midx_v.at[pl.ds(half, half)]], buf_b, sem_b)
        ma.wait()
        write_sel(buf_a, msel_v, 0)
        oa = pltpu.async_copy(buf_a, mout.at[pl.ds(base, half)], sem_oa)
        mb.wait()
        write_sel(buf_b, msel_v, half)
        ob = pltpu.async_copy(buf_b, mout.at[pl.ds(base + half, half)], sem_ob)
        oa.wait()
        ua = pltpu.async_copy(utab.at[uidx_v.at[pl.ds(0, half)]], buf_a, sem_a)
        ob.wait()
        ub = pltpu.async_copy(utab.at[uidx_v.at[pl.ds(half, half)]], buf_b, sem_b)
        ua.wait()
        write_sel(buf_a, usel_v, 0)
        oa = pltpu.async_copy(buf_a, uout.at[pl.ds(base, half)], sem_oa)
        ub.wait()
        write_sel(buf_b, usel_v, half)
        ob = pltpu.async_copy(buf_b, uout.at[pl.ds(base + half, half)], sem_ob)
        oa.wait()
        ob.wait()

    return gather_k(mt128, ut128, mids, uids)


_GROUP = 50176


def _prep_body(m0, m1, u0, u1, eyes, mo, uo):
    f32 = jnp.float32
    dims = (((0,), (0,)), ((), ()))

    def pack(a, b):
        return (jax.lax.dot_general(a[...], eyes[0, 0], dims,
                                    preferred_element_type=f32)
                + jax.lax.dot_general(b[...], eyes[0, 1], dims,
                                      preferred_element_type=f32))

    mo[...] = pack(m0, m1)
    uo[...] = pack(u0, u1)


def _prep(movieT, userT, eyes):
    C = 7168
    nb = _GROUP // C
    grid = (nb,)

    def tblock(q):
        return lambda i: (0, q * nb + i)

    return pl.pallas_call(
        _prep_body,
        grid=grid,
        in_specs=(
            [pl.BlockSpec((movieT.shape[0], C), tblock(q)) for q in range(2)]
            + [pl.BlockSpec((userT.shape[0], C), tblock(q)) for q in range(2)]
            + [pl.BlockSpec((1, 2) + eyes.shape[2:], lambda i: (0, 0, 0, 0))]
        ),
        out_specs=[
            pl.BlockSpec((C, _PAD), lambda i: (i, 0)),
            pl.BlockSpec((C, _PAD), lambda i: (i, 0)),
        ],
        out_shape=[
            jax.ShapeDtypeStruct((_GROUP, _PAD), jnp.float32),
            jax.ShapeDtypeStruct((_GROUP, _PAD), jnp.float32),
        ],
    )(movieT, movieT, userT, userT, eyes)


def _mlp_body(mv, us, gnT, vt, wg, bg, w1m, w1u, w1g, w1v, b1,
              w2, b2, w3, b3, w4, b4, out):
    f32 = jnp.float32
    emb = w1m.shape[0]
    dims = (((0,), (0,)), ((), ()))
    g = jax.lax.dot_general(gnT[...], wg[...], dims,
                            preferred_element_type=f32) + bg[...]
    g = jnp.maximum(g, 0.0)
    mvr = mv[...]
    usr = us[...]
    mve = jnp.where(mvr[:, 120:121] < 0.5, mvr[:, 0:emb], mvr[:, 64:64 + emb])
    use = jnp.where(usr[:, 120:121] < 0.5, usr[:, 0:emb], usr[:, 64:64 + emb])
    x = (jnp.dot(mve, w1m[...], preferred_element_type=f32)
         + jnp.dot(use, w1u[...], preferred_element_type=f32)
         + jnp.dot(g, w1g[...], preferred_element_type=f32)
         + jax.lax.dot_general(vt[...], w1v[...], dims,
                               preferred_element_type=f32)
         + b1[...])
    x = jnp.maximum(x, 0.0)
    x = jnp.maximum(jnp.dot(x, w2[...], preferred_element_type=f32) + b2[...], 0.0)
    x = jnp.maximum(jnp.dot(x, w3[...], preferred_element_type=f32) + b3[...], 0.0)
    dt = (((0,), (1,)), ((), ()))
    xt = jnp.maximum(jax.lax.dot_general(w4[...], x, dt,
                                         preferred_element_type=f32)
                     + b4[...], 0.0)
    m = jnp.max(xt, axis=0, keepdims=True)
    e = jnp.exp(xt - m)
    out[...] = e / jnp.sum(e, axis=0, keepdims=True)


def _mlp(movieE, userE, genreT, vote, Wg, bg, W1m, W1u, W1g,
         w1v, b1, W2, b2, W3, b3, W4, b4):
    B = movieE.shape[0]
    T = 4096
    grid = (B // T,)

    def btile(minor):
        return pl.BlockSpec((T, minor), lambda i: (i, 0))

    def full(a):
        return pl.BlockSpec(a.shape, lambda i: (0, 0))

    return pl.pallas_call(
        _mlp_body,
        grid=grid,
        in_specs=[
            btile(movieE.shape[1]),
            btile(userE.shape[1]),
            pl.BlockSpec((genreT.shape[0], T), lambda i: (0, i)),
            pl.BlockSpec((1, T), lambda i: (0, i)),
            full(Wg), full(bg), full(W1m), full(W1u), full(W1g),
            full(w1v), full(b1), full(W2), full(b2), full(W3), full(b3),
            full(W4), full(b4),
        ],
        out_specs=pl.BlockSpec((5, T), lambda i: (0, i)),
        out_shape=jax.ShapeDtypeStruct((5, B), jnp.float32),
    )(movieE, userE, genreT, vote, Wg, bg, W1m, W1u, W1g, w1v,
      b1, W2, b2, W3, b3, W4, b4)


def kernel(userId, movieId, genre, vote_average, release_date, movie_table,
           user_table, Wg, bg, W1, b1, W2, b2, W3, b3, W4, b4):
    B = userId.shape[0]
    emb = movie_table.shape[1]
    mids = movieId.reshape(B)
    uids = userId.reshape(B)
    eyes = jnp.stack([jnp.eye(emb, _PAD, k=64 * q, dtype=jnp.float32)
                      for q in range(2)])[None]
    mt128, ut128 = _prep(movie_table.T, user_table.T, eyes)
    movieE, userE = _sc_gather(mt128, ut128, mids, uids)
    genreT = genre.reshape(B, genre.shape[-1]).T
    W1m = W1[0:20]
    W1u = W1[20:40]
    W1g = W1[40:48]
    w1v = W1[48:49]
    return _mlp(movieE, userE, genreT, vote_average.T,
                Wg, bg.reshape(1, -1), W1m, W1u, W1g, w1v, b1.reshape(1, -1),
                W2, b2.reshape(1, -1), W3, b3.reshape(1, -1),
                W4, b4.reshape(-1, 1)).T
